# baseline (speedup 1.0000x reference)
import functools

import jax
import jax.numpy as jnp
from jax import lax
from jax.experimental import pallas as pl
from jax.experimental.pallas import tpu as pltpu
from jax.experimental.pallas import tpu_sc as plsc

N = 10000
E = 160000
D = 256
H = 128
NC = 2
NS = 16
_MESH = plsc.VectorSubcoreMesh(core_axis_name="c", subcore_axis_name="s")

ROWS_CH = 624


def _striped_rows(s, fn):
    fn(pl.multiple_of(s * ROWS_CH, 8), ROWS_CH)

    @pl.when(s == NS - 1)
    def _():
        fn(NS * ROWS_CH, N - NS * ROWS_CH)


NP = 10240
EDGES_T = 4992


@functools.partial(
    pl.kernel,
    out_type=jax.ShapeDtypeStruct((NC, NP), jnp.float32),
    mesh=_MESH,
    scratch_types=[
        pltpu.VMEM((E - (NC * NS - 1) * EDGES_T,), jnp.int32),
        pltpu.VMEM((NP,), jnp.float32),
        pltpu.VMEM((NS, 640), jnp.float32),
        pltpu.VMEM((640,), jnp.float32),
        pltpu.VMEM_SHARED((NS, NP), jnp.float32),
    ],
    compiler_params=pltpu.CompilerParams(needs_layout_passes=False),
)
def _sc_deg(dst_hbm, zeros_hbm, deg_out, idx_v, deg_v, red_v, sum_v, slots_sh):
    c = lax.axis_index("c")
    s = lax.axis_index("s")
    tile = c * NS + s

    pltpu.sync_copy(zeros_hbm, deg_v)
    base = pl.multiple_of(tile * EDGES_T, 8)
    last = E - (NC * NS - 1) * EDGES_T

    @pl.when(tile < NC * NS - 1)
    def _():
        pltpu.sync_copy(dst_hbm.at[pl.ds(base, EDGES_T)],
                        idx_v.at[pl.ds(0, EDGES_T)])

    @pl.when(tile == NC * NS - 1)
    def _():
        pltpu.sync_copy(dst_hbm.at[pl.ds(base, last)], idx_v)

    ones = jnp.full((16,), 1.0, jnp.float32)

    def body(i, _):
        iv = idx_v[pl.ds(i * 16, 16)]
        plsc.addupdate_scatter(deg_v, [iv], ones)
        return _

    n_groups = jnp.where(tile == NC * NS - 1, last // 16, EDGES_T // 16)
    lax.fori_loop(0, n_groups, body, None)

    pltpu.sync_copy(deg_v, slots_sh.at[s])
    plsc.subcore_barrier()
    col = pl.multiple_of(s * 640, 8)
    pltpu.sync_copy(slots_sh.at[:, pl.ds(col, 640)], red_v)
    for r in range(640 // 16):
        acc = red_v[0, pl.ds(r * 16, 16)]
        for k in range(1, NS):
            acc = acc + red_v[k, pl.ds(r * 16, 16)]
        sum_v[pl.ds(r * 16, 16)] = acc
    pltpu.sync_copy(sum_v, deg_out.at[c, pl.ds(col, 640)])


CH = 100
K_AGG = 100


@functools.partial(
    pl.kernel,
    out_type=jax.ShapeDtypeStruct((N, D), jnp.float32),
    mesh=_MESH,
    scratch_types=[
        pltpu.VMEM((K_AGG,), jnp.int32),
        pltpu.VMEM((K_AGG,), jnp.int32),
        pltpu.VMEM((K_AGG,), jnp.int32),
        pltpu.VMEM((K_AGG,), jnp.int32),
        pltpu.VMEM((K_AGG, H), jnp.float32),
        pltpu.VMEM((K_AGG, H), jnp.float32),
        pltpu.VMEM_SHARED((N, H), jnp.float32),
        pltpu.SemaphoreType.DMA,
        pltpu.SemaphoreType.DMA,
        pltpu.SemaphoreType.DMA,
        pltpu.SemaphoreType.DMA,
        pltpu.SemaphoreType.DMA,
        pltpu.SemaphoreType.DMA,
        pltpu.SemaphoreType.DMA,
        pltpu.SemaphoreType.DMA,
    ],
)
def _sc_agg(src_hbm, dst_hbm, g0_hbm, g1_hbm, out_hbm,
            isrc0, isrc1, idst0, idst1, rows0, rows1, acc_sh,
            sg0, sg1, ss0, ss1, sis0, sis1, sid0, sid1):
    c = lax.axis_index("c")
    s = lax.axis_index("s")

    def isrc_start(j, buf, sem):
        pltpu.async_copy(src_hbm.at[s, j], buf, sem)

    def isrc_wait(j, buf, sem):
        pltpu.make_async_copy(src_hbm.at[s, j], buf, sem).wait()

    def idst_start(j, buf, sem):
        pltpu.async_copy(dst_hbm.at[s, j], buf, sem)

    def idst_wait(j, buf, sem):
        pltpu.make_async_copy(dst_hbm.at[s, j], buf, sem).wait()

    def g_start(ibuf, buf, sem):
        @pl.when(c == 0)
        def _():
            pltpu.async_copy(g0_hbm.at[ibuf], buf, sem)

        @pl.when(c == 1)
        def _():
            pltpu.async_copy(g1_hbm.at[ibuf], buf, sem)

    def g_wait(ibuf, buf, sem):
        pltpu.make_async_copy(g0_hbm.at[ibuf], buf, sem).wait()

    def s_start(ibuf, buf, sem):
        pltpu.async_copy(buf, acc_sh.at[ibuf], sem, add=True)

    def s_wait(ibuf, buf, sem):
        pltpu.make_async_copy(buf, acc_sh.at[ibuf], sem).wait()

    pltpu.sync_copy(src_hbm.at[s, 0], isrc0)
    pltpu.sync_copy(dst_hbm.at[s, 0], idst0)
    pltpu.sync_copy(src_hbm.at[s, 1], isrc1)
    pltpu.sync_copy(dst_hbm.at[s, 1], idst1)
    g_start(isrc0, rows0, sg0)
    g_start(isrc1, rows1, sg1)

    def _init(off, sz):
        @pl.when(c == 0)
        def _():
            pltpu.sync_copy(g0_hbm.at[pl.ds(off, sz)],
                            acc_sh.at[pl.ds(off, sz)])

        @pl.when(c == 1)
        def _():
            pltpu.sync_copy(g1_hbm.at[pl.ds(off, sz)],
                            acc_sh.at[pl.ds(off, sz)])

    _striped_rows(s, _init)
    plsc.subcore_barrier()

    def body(p, _):
        j0 = p * 2
        g_wait(isrc0, rows0, sg0)
        s_start(idst0, rows0, ss0)
        isrc_start(j0 + 2, isrc0, sis0)
        g_wait(isrc1, rows1, sg1)
        s_wait(idst0, rows0, ss0)
        idst_start(j0 + 2, idst0, sid0)
        isrc_wait(j0 + 2, isrc0, sis0)
        g_start(isrc0, rows0, sg0)
        s_start(idst1, rows1, ss1)
        isrc_start(j0 + 3, isrc1, sis1)
        s_wait(idst1, rows1, ss1)
        idst_start(j0 + 3, idst1, sid1)
        isrc_wait(j0 + 3, isrc1, sis1)
        g_start(isrc1, rows1, sg1)
        idst_wait(j0 + 2, idst0, sid0)
        idst_wait(j0 + 3, idst1, sid1)
        return _

    lax.fori_loop(0, CH // 2 - 1, body, None)

    g_wait(isrc0, rows0, sg0)
    s_start(idst0, rows0, ss0)
    g_wait(isrc1, rows1, sg1)
    s_wait(idst0, rows0, ss0)
    s_start(idst1, rows1, ss1)
    s_wait(idst1, rows1, ss1)

    plsc.subcore_barrier()

    _striped_rows(s, lambda off, sz: pltpu.sync_copy(
        acc_sh.at[pl.ds(off, sz)],
        out_hbm.at[pl.ds(off, sz), pl.ds(c * H, H)]))


def _tc_lin_body(x_ref, w_ref, d0_ref, d1_ref, g0_ref, g1_ref, dv_ref):
    h = jnp.dot(x_ref[...], w_ref[...], preferred_element_type=jnp.float32)
    deg = d0_ref[...] + d1_ref[...] + 1.0
    dinv = lax.rsqrt(deg)
    g0_ref[...] = h[:, :H] * dinv
    g1_ref[...] = h[:, H:] * dinv
    dv_ref[...] = dinv


def _tc_lin(x, W, d0, d1):
    blk = 2000
    grid = (N // blk,)
    return pl.pallas_call(
        _tc_lin_body,
        grid=grid,
        in_specs=[
            pl.BlockSpec((blk, D), lambda i: (i, 0)),
            pl.BlockSpec((D, D), lambda i: (0, 0)),
            pl.BlockSpec((blk, 1), lambda i: (i, 0)),
            pl.BlockSpec((blk, 1), lambda i: (i, 0)),
        ],
        out_specs=[
            pl.BlockSpec((blk, H), lambda i: (i, 0)),
            pl.BlockSpec((blk, H), lambda i: (i, 0)),
            pl.BlockSpec((blk, 1), lambda i: (i, 0)),
        ],
        out_shape=[
            jax.ShapeDtypeStruct((N, H), jnp.float32),
            jax.ShapeDtypeStruct((N, H), jnp.float32),
            jax.ShapeDtypeStruct((N, 1), jnp.float32),
        ],
    )(x, W, d0, d1)


def _tc_out_body(acc_ref, dv_ref, b_ref, o_ref):
    o_ref[...] = acc_ref[...] * dv_ref[...] + b_ref[...]


def _tc_out(acc, dinv, b):
    blk = 1000
    grid = (N // blk,)
    return pl.pallas_call(
        _tc_out_body,
        grid=grid,
        in_specs=[
            pl.BlockSpec((blk, D), lambda i: (i, 0)),
            pl.BlockSpec((blk, 1), lambda i: (i, 0)),
            pl.BlockSpec((1, D), lambda i: (0, 0)),
        ],
        out_specs=pl.BlockSpec((blk, D), lambda i: (i, 0)),
        out_shape=jax.ShapeDtypeStruct((N, D), jnp.float32),
    )(acc, dinv, b)


def kernel(x, edge_index, W, b):
    src = edge_index[0]
    dst = edge_index[1]
    zeros = jnp.zeros((NP,), jnp.float32)
    deg2 = _sc_deg(dst, zeros)
    d0 = deg2[0, :N].reshape(N, 1)
    d1 = deg2[1, :N].reshape(N, 1)
    g0, g1, dinv = _tc_lin(x, W, d0, d1)
    src3 = src.reshape(NS, CH, K_AGG)
    dst3 = dst.reshape(NS, CH, K_AGG)
    acc = _sc_agg(src3, dst3, g0, g1)
    return _tc_out(acc, dinv, b.reshape(1, D))


# speedup vs baseline: 1.0238x; 1.0238x over previous
import functools

import jax
import jax.numpy as jnp
from jax import lax
from jax.experimental import pallas as pl
from jax.experimental.pallas import tpu as pltpu
from jax.experimental.pallas import tpu_sc as plsc

N = 10000
E = 160000
D = 256
H = 128
NC = 2
NS = 16
_MESH = plsc.VectorSubcoreMesh(core_axis_name="c", subcore_axis_name="s")

ROWS_CH = 624


def _striped_rows(s, fn):
    fn(pl.multiple_of(s * ROWS_CH, 8), ROWS_CH)

    @pl.when(s == NS - 1)
    def _():
        fn(NS * ROWS_CH, N - NS * ROWS_CH)


NP = 10240
EDGES_T = 4992


@functools.partial(
    pl.kernel,
    out_type=jax.ShapeDtypeStruct((NC, NP), jnp.float32),
    mesh=_MESH,
    scratch_types=[
        pltpu.VMEM((E - (NC * NS - 1) * EDGES_T,), jnp.int32),
        pltpu.VMEM((NP,), jnp.float32),
        pltpu.VMEM((NS, 640), jnp.float32),
        pltpu.VMEM((640,), jnp.float32),
        pltpu.VMEM_SHARED((NS, NP), jnp.float32),
    ],
    compiler_params=pltpu.CompilerParams(needs_layout_passes=False),
)
def _sc_deg(dst_hbm, zeros_hbm, deg_out, idx_v, deg_v, red_v, sum_v, slots_sh):
    c = lax.axis_index("c")
    s = lax.axis_index("s")
    tile = c * NS + s

    pltpu.sync_copy(zeros_hbm, deg_v)
    base = pl.multiple_of(tile * EDGES_T, 8)
    last = E - (NC * NS - 1) * EDGES_T

    @pl.when(tile < NC * NS - 1)
    def _():
        pltpu.sync_copy(dst_hbm.at[pl.ds(base, EDGES_T)],
                        idx_v.at[pl.ds(0, EDGES_T)])

    @pl.when(tile == NC * NS - 1)
    def _():
        pltpu.sync_copy(dst_hbm.at[pl.ds(base, last)], idx_v)

    ones = jnp.full((16,), 1.0, jnp.float32)

    def body(i, _):
        iv = idx_v[pl.ds(i * 16, 16)]
        plsc.addupdate_scatter(deg_v, [iv], ones)
        return _

    n_groups = jnp.where(tile == NC * NS - 1, last // 16, EDGES_T // 16)
    lax.fori_loop(0, n_groups, body, None)

    pltpu.sync_copy(deg_v, slots_sh.at[s])
    plsc.subcore_barrier()
    col = pl.multiple_of(s * 640, 8)
    pltpu.sync_copy(slots_sh.at[:, pl.ds(col, 640)], red_v)
    for r in range(640 // 16):
        acc = red_v[0, pl.ds(r * 16, 16)]
        for k in range(1, NS):
            acc = acc + red_v[k, pl.ds(r * 16, 16)]
        sum_v[pl.ds(r * 16, 16)] = acc
    pltpu.sync_copy(sum_v, deg_out.at[c, pl.ds(col, 640)])


CH = 80
K_AGG = 125


@functools.partial(
    pl.kernel,
    out_type=jax.ShapeDtypeStruct((N, D), jnp.float32),
    mesh=_MESH,
    scratch_types=[
        pltpu.VMEM((K_AGG,), jnp.int32),
        pltpu.VMEM((K_AGG,), jnp.int32),
        pltpu.VMEM((K_AGG,), jnp.int32),
        pltpu.VMEM((K_AGG,), jnp.int32),
        pltpu.VMEM((K_AGG, H), jnp.float32),
        pltpu.VMEM((K_AGG, H), jnp.float32),
        pltpu.VMEM_SHARED((N, H), jnp.float32),
        pltpu.SemaphoreType.DMA,
        pltpu.SemaphoreType.DMA,
        pltpu.SemaphoreType.DMA,
        pltpu.SemaphoreType.DMA,
        pltpu.SemaphoreType.DMA,
        pltpu.SemaphoreType.DMA,
        pltpu.SemaphoreType.DMA,
        pltpu.SemaphoreType.DMA,
    ],
)
def _sc_agg(src_hbm, dst_hbm, g0_hbm, g1_hbm, out_hbm,
            isrc0, isrc1, idst0, idst1, rows0, rows1, acc_sh,
            sg0, sg1, ss0, ss1, sis0, sis1, sid0, sid1):
    c = lax.axis_index("c")
    s = lax.axis_index("s")

    def isrc_start(j, buf, sem):
        pltpu.async_copy(src_hbm.at[s, j], buf, sem)

    def isrc_wait(j, buf, sem):
        pltpu.make_async_copy(src_hbm.at[s, j], buf, sem).wait()

    def idst_start(j, buf, sem):
        pltpu.async_copy(dst_hbm.at[s, j], buf, sem)

    def idst_wait(j, buf, sem):
        pltpu.make_async_copy(dst_hbm.at[s, j], buf, sem).wait()

    def g_start(ibuf, buf, sem):
        @pl.when(c == 0)
        def _():
            pltpu.async_copy(g0_hbm.at[ibuf], buf, sem)

        @pl.when(c == 1)
        def _():
            pltpu.async_copy(g1_hbm.at[ibuf], buf, sem)

    def g_wait(ibuf, buf, sem):
        pltpu.make_async_copy(g0_hbm.at[ibuf], buf, sem).wait()

    def s_start(ibuf, buf, sem):
        pltpu.async_copy(buf, acc_sh.at[ibuf], sem, add=True)

    def s_wait(ibuf, buf, sem):
        pltpu.make_async_copy(buf, acc_sh.at[ibuf], sem).wait()

    pltpu.sync_copy(src_hbm.at[s, 0], isrc0)
    pltpu.sync_copy(dst_hbm.at[s, 0], idst0)
    pltpu.sync_copy(src_hbm.at[s, 1], isrc1)
    pltpu.sync_copy(dst_hbm.at[s, 1], idst1)
    g_start(isrc0, rows0, sg0)
    g_start(isrc1, rows1, sg1)

    def _init(off, sz):
        @pl.when(c == 0)
        def _():
            pltpu.sync_copy(g0_hbm.at[pl.ds(off, sz)],
                            acc_sh.at[pl.ds(off, sz)])

        @pl.when(c == 1)
        def _():
            pltpu.sync_copy(g1_hbm.at[pl.ds(off, sz)],
                            acc_sh.at[pl.ds(off, sz)])

    _striped_rows(s, _init)
    plsc.subcore_barrier()

    def body(p, _):
        j0 = p * 2
        g_wait(isrc0, rows0, sg0)
        s_start(idst0, rows0, ss0)
        isrc_start(j0 + 2, isrc0, sis0)
        g_wait(isrc1, rows1, sg1)
        s_wait(idst0, rows0, ss0)
        idst_start(j0 + 2, idst0, sid0)
        isrc_wait(j0 + 2, isrc0, sis0)
        g_start(isrc0, rows0, sg0)
        s_start(idst1, rows1, ss1)
        isrc_start(j0 + 3, isrc1, sis1)
        s_wait(idst1, rows1, ss1)
        idst_start(j0 + 3, idst1, sid1)
        isrc_wait(j0 + 3, isrc1, sis1)
        g_start(isrc1, rows1, sg1)
        idst_wait(j0 + 2, idst0, sid0)
        idst_wait(j0 + 3, idst1, sid1)
        return _

    lax.fori_loop(0, CH // 2 - 1, body, None)

    g_wait(isrc0, rows0, sg0)
    s_start(idst0, rows0, ss0)
    g_wait(isrc1, rows1, sg1)
    s_wait(idst0, rows0, ss0)
    s_start(idst1, rows1, ss1)
    s_wait(idst1, rows1, ss1)

    plsc.subcore_barrier()

    _striped_rows(s, lambda off, sz: pltpu.sync_copy(
        acc_sh.at[pl.ds(off, sz)],
        out_hbm.at[pl.ds(off, sz), pl.ds(c * H, H)]))


def _tc_lin_body(x_ref, w_ref, d0_ref, d1_ref, g0_ref, g1_ref, dv_ref):
    h = jnp.dot(x_ref[...], w_ref[...], preferred_element_type=jnp.float32)
    deg = d0_ref[...] + d1_ref[...] + 1.0
    dinv = lax.rsqrt(deg)
    g0_ref[...] = h[:, :H] * dinv
    g1_ref[...] = h[:, H:] * dinv
    dv_ref[...] = dinv


def _tc_lin(x, W, d0, d1):
    blk = 2000
    grid = (N // blk,)
    return pl.pallas_call(
        _tc_lin_body,
        grid=grid,
        in_specs=[
            pl.BlockSpec((blk, D), lambda i: (i, 0)),
            pl.BlockSpec((D, D), lambda i: (0, 0)),
            pl.BlockSpec((blk, 1), lambda i: (i, 0)),
            pl.BlockSpec((blk, 1), lambda i: (i, 0)),
        ],
        out_specs=[
            pl.BlockSpec((blk, H), lambda i: (i, 0)),
            pl.BlockSpec((blk, H), lambda i: (i, 0)),
            pl.BlockSpec((blk, 1), lambda i: (i, 0)),
        ],
        out_shape=[
            jax.ShapeDtypeStruct((N, H), jnp.float32),
            jax.ShapeDtypeStruct((N, H), jnp.float32),
            jax.ShapeDtypeStruct((N, 1), jnp.float32),
        ],
    )(x, W, d0, d1)


def _tc_out_body(acc_ref, dv_ref, b_ref, o_ref):
    o_ref[...] = acc_ref[...] * dv_ref[...] + b_ref[...]


def _tc_out(acc, dinv, b):
    blk = 1000
    grid = (N // blk,)
    return pl.pallas_call(
        _tc_out_body,
        grid=grid,
        in_specs=[
            pl.BlockSpec((blk, D), lambda i: (i, 0)),
            pl.BlockSpec((blk, 1), lambda i: (i, 0)),
            pl.BlockSpec((1, D), lambda i: (0, 0)),
        ],
        out_specs=pl.BlockSpec((blk, D), lambda i: (i, 0)),
        out_shape=jax.ShapeDtypeStruct((N, D), jnp.float32),
    )(acc, dinv, b)


def kernel(x, edge_index, W, b):
    src = edge_index[0]
    dst = edge_index[1]
    zeros = jnp.zeros((NP,), jnp.float32)
    deg2 = _sc_deg(dst, zeros)
    d0 = deg2[0, :N].reshape(N, 1)
    d1 = deg2[1, :N].reshape(N, 1)
    g0, g1, dinv = _tc_lin(x, W, d0, d1)
    src3 = src.reshape(NS, CH, K_AGG)
    dst3 = dst.reshape(NS, CH, K_AGG)
    acc = _sc_agg(src3, dst3, g0, g1)
    return _tc_out(acc, dinv, b.reshape(1, D))
